# Initial kernel scaffold; baseline (speedup 1.0000x reference)
import jax
import jax.numpy as jnp
from jax.experimental import pallas as pl


def kernel(input_features, edges, W1, a1, R1, W2, a2, W3, a3, R3):
    raise NotImplementedError("write your pallas kernel here")


# speedup vs baseline: 4.8922x; 4.8922x over previous
import functools

import jax
import jax.numpy as jnp
from jax import lax
from jax.experimental import pallas as pl
from jax.experimental.pallas import tpu as pltpu
from jax.experimental.pallas import tpu_sc as plsc

N_NODES = 10000
N_EDGES = 320000
D_FEAT = 128
UNITS = 128
H12 = 4
H3 = 6
OUT_DIM = 121

ACC_W = 128
NW = 32
EDGES_PER_W = N_EDGES // NW
CHUNK = 16
EBLK = 2000
NBLK = EDGES_PER_W // EBLK
BLK_CHUNKS = EBLK // CHUNK
ACC_N = 10240
STRIPE = ACC_N // 16
ZROWS = 32

_HIGH = jax.lax.Precision.HIGHEST


def _dot(a, b):
    return jnp.dot(a, b, preferred_element_type=jnp.float32, precision=_HIGH)


def _make_sc_edge(num_heads):
    H = num_heads
    mesh = plsc.VectorSubcoreMesh(core_axis_name="c", subcore_axis_name="s")

    @functools.partial(
        pl.kernel,
        mesh=mesh,
        compiler_params=pltpu.CompilerParams(
            needs_layout_passes=False, use_tc_tiling_on_sc=False),
        out_type=(
            jax.ShapeDtypeStruct((H, 2, N_NODES, ACC_W), jnp.float32),
            jax.ShapeDtypeStruct((H, 2, 16, ACC_N), jnp.float32),
        ),
        scratch_types=[
            pltpu.VMEM((EBLK,), jnp.int32),
            pltpu.VMEM((EBLK,), jnp.int32),
            pltpu.VMEM((N_NODES,), jnp.float32),
            pltpu.VMEM((N_NODES,), jnp.float32),
            pltpu.VMEM((CHUNK, 128), jnp.float32),
            pltpu.VMEM((CHUNK, ACC_W), jnp.float32),
            pltpu.VMEM((CHUNK,), jnp.float32),
            pltpu.VMEM((ZROWS, ACC_W), jnp.float32),
            pltpu.VMEM((ACC_N,), jnp.float32),
            pltpu.VMEM_SHARED((ACC_N, ACC_W), jnp.float32),
        ],
    )
    def sc_edge(h_hbm, st_hbm, tgt_hbm, src_hbm, out_hbm, den_hbm,
                tgt_v, src_v, st_v, ss_v, grows, rowsbuf, wbuf, zbuf,
                denom_v, acc):
        cid = lax.axis_index("c")
        sid = lax.axis_index("s")
        wid = sid * 2 + cid
        row0 = sid * STRIPE
        zero16 = jnp.zeros((16,), jnp.float32)

        for r in range(ZROWS):
            for j in range(ACC_W // 16):
                zbuf[r, pl.ds(j * 16, 16)] = zero16

        ebase = wid * EDGES_PER_W

        def head_body(h, hcarry):
            pltpu.sync_copy(st_hbm.at[h], st_v)
            pltpu.sync_copy(st_hbm.at[H + h], ss_v)
            for kk in range(STRIPE // ZROWS):
                pltpu.sync_copy(zbuf, acc.at[pl.ds(row0 + kk * ZROWS, ZROWS)])

            def zbody(c, carry):
                denom_v[pl.ds(c * 16, 16)] = zero16
                return carry

            lax.fori_loop(0, ACC_N // 16, zbody, 0)
            plsc.subcore_barrier()

            hoff = h * N_NODES

            def body(c, carry):
                base = c * CHUNK
                it = tgt_v[pl.ds(base, CHUNK)]
                isrc = src_v[pl.ds(base, CHUNK)]
                st = plsc.load_gather(st_v, [it])
                ss = plsc.load_gather(ss_v, [isrc])
                e = st + ss
                e = jnp.where(e >= 0.0, e, 0.2 * e)
                e = jnp.minimum(jnp.maximum(e, -2.0), 2.0)
                w = jnp.exp(e)
                wbuf[...] = w
                pltpu.sync_copy(h_hbm.at[isrc + hoff], grows)
                plsc.addupdate_scatter(denom_v, [it], w)

                def scale_body(i, icarry):
                    wi = plsc.load_gather(
                        wbuf, [jnp.full((16,), 0, jnp.int32) + i])
                    for j in range(8):
                        rowsbuf[i, pl.ds(j * 16, 16)] = (
                            grows[i, pl.ds(j * 16, 16)] * wi)
                    return icarry

                lax.fori_loop(0, CHUNK, scale_body, 0)
                pltpu.sync_copy(rowsbuf, acc.at[it], add=True)
                return carry

            def blk_body(blk, carry):
                pltpu.sync_copy(
                    tgt_hbm.at[pl.ds(ebase + blk * EBLK, EBLK)], tgt_v)
                pltpu.sync_copy(
                    src_hbm.at[pl.ds(ebase + blk * EBLK, EBLK)], src_v)
                lax.fori_loop(0, BLK_CHUNKS, body, 0)
                return carry

            lax.fori_loop(0, NBLK, blk_body, 0)
            plsc.subcore_barrier()
            pltpu.sync_copy(denom_v, den_hbm.at[h, cid, sid])

            @pl.when(sid < 15)
            def _():
                pltpu.sync_copy(
                    acc.at[pl.ds(row0, STRIPE)],
                    out_hbm.at[h, cid, pl.ds(row0, STRIPE)])

            @pl.when(sid == 15)
            def _():
                pltpu.sync_copy(
                    acc.at[pl.ds(row0, N_NODES - 15 * STRIPE)],
                    out_hbm.at[h, cid, pl.ds(row0, N_NODES - 15 * STRIPE)])

            plsc.subcore_barrier()
            return hcarry

        lax.fori_loop(0, H, head_body, 0)

    return sc_edge


_sc_edge_4 = _make_sc_edge(H12)
_sc_edge_6 = _make_sc_edge(H3)


NB = 10
BR = N_NODES // NB


def _norm_heads(acc, den):
    num = acc[:, 0] + acc[:, 1]
    d = jnp.sum(den, axis=(2, 3))
    return num / jnp.maximum(d, 1e-20)[..., None]


def _norm_concat(acc, den, res):
    hh = _norm_heads(acc, den)
    cat = jnp.concatenate([hh[i] for i in range(hh.shape[0])], axis=-1)
    return jnp.maximum(cat + res, 0.0)


def _tck1_body(x_ref, wcat_ref, b_ref, r_ref, hflat_ref, s_ref, xr_ref):
    x = x_ref[...]
    y = _dot(x, wcat_ref[...])
    for h in range(H12):
        hflat_ref[h] = y[:, h * 128:(h + 1) * 128]
    s_ref[...] = _dot(x, b_ref[...])
    xr_ref[...] = _dot(x, r_ref[...])


def _tck2_body(acc_ref, den_ref, xr_ref, wcat_ref, b_ref,
               hflat_ref, s_ref, h1_ref):
    h1 = _norm_concat(acc_ref[...], den_ref[...], xr_ref[...])
    y = _dot(h1, wcat_ref[...])
    for h in range(H12):
        hflat_ref[h] = y[:, h * 128:(h + 1) * 128]
    s_ref[...] = _dot(h1, b_ref[...])
    h1_ref[...] = h1


def _tck3_body(acc_ref, den_ref, h1_ref, wcat_ref, b_ref, r3_ref,
               hflat_ref, s_ref, hr_ref):
    h2 = _norm_concat(acc_ref[...], den_ref[...], h1_ref[...])
    y = _dot(h2, wcat_ref[...])
    for h in range(H3):
        hflat_ref[h] = y[:, h * 128:(h + 1) * 128]
    s_ref[...] = _dot(h2, b_ref[...])
    hr_ref[...] = _dot(h2, r3_ref[...])


def _tck4_body(acc_ref, den_ref, hr_ref, out_ref):
    hh = _norm_heads(acc_ref[...], den_ref[...])
    out_ref[...] = jnp.mean(hh, axis=0) + hr_ref[...]


def _row_spec(shape_tail):
    return pl.BlockSpec((BR,) + shape_tail, lambda i: (i,) + (0,) * len(shape_tail))


def _full_spec(shape):
    return pl.BlockSpec(shape, lambda i: (0,) * len(shape))


def _hflat_spec(H):
    return pl.BlockSpec((H, BR, 128), lambda i: (0, i, 0))


def _acc_spec(H):
    return pl.BlockSpec((H, 2, BR, ACC_W), lambda i: (0, 0, i, 0))


def _den_spec(H):
    return pl.BlockSpec((H, BR, 2, 16), lambda i: (0, i, 0, 0))


def _tck1(x, wcat, b, r):
    return pl.pallas_call(
        _tck1_body,
        grid=(NB,),
        in_specs=[_row_spec((D_FEAT,)), _full_spec(wcat.shape),
                  _full_spec(b.shape), _full_spec(r.shape)],
        out_specs=[_hflat_spec(H12), _row_spec((2 * H12,)),
                   _row_spec((H12 * UNITS,))],
        out_shape=[
            jax.ShapeDtypeStruct((H12, N_NODES, 128), jnp.float32),
            jax.ShapeDtypeStruct((N_NODES, 2 * H12), jnp.float32),
            jax.ShapeDtypeStruct((N_NODES, H12 * UNITS), jnp.float32),
        ],
    )(x, wcat, b, r)


def _tck2(acc, den, xr, wcat, b):
    return pl.pallas_call(
        _tck2_body,
        grid=(NB,),
        in_specs=[_acc_spec(H12), _den_spec(H12), _row_spec((H12 * UNITS,)),
                  _full_spec(wcat.shape), _full_spec(b.shape)],
        out_specs=[_hflat_spec(H12), _row_spec((2 * H12,)),
                   _row_spec((H12 * UNITS,))],
        out_shape=[
            jax.ShapeDtypeStruct((H12, N_NODES, 128), jnp.float32),
            jax.ShapeDtypeStruct((N_NODES, 2 * H12), jnp.float32),
            jax.ShapeDtypeStruct((N_NODES, H12 * UNITS), jnp.float32),
        ],
    )(acc, den, xr, wcat, b)


def _tck3(acc, den, h1, wcat, b, r3p):
    return pl.pallas_call(
        _tck3_body,
        grid=(NB,),
        in_specs=[_acc_spec(H12), _den_spec(H12), _row_spec((H12 * UNITS,)),
                  _full_spec(wcat.shape), _full_spec(b.shape),
                  _full_spec(r3p.shape)],
        out_specs=[_hflat_spec(H3), _row_spec((2 * H3,)),
                   _row_spec((128,))],
        out_shape=[
            jax.ShapeDtypeStruct((H3, N_NODES, 128), jnp.float32),
            jax.ShapeDtypeStruct((N_NODES, 2 * H3), jnp.float32),
            jax.ShapeDtypeStruct((N_NODES, 128), jnp.float32),
        ],
    )(acc, den, h1, wcat, b, r3p)


def _tck4(acc, den, hr):
    return pl.pallas_call(
        _tck4_body,
        grid=(NB,),
        in_specs=[_acc_spec(H3), _den_spec(H3), _row_spec((128,))],
        out_specs=pl.BlockSpec((BR, 128), lambda i: (i, 0)),
        out_shape=jax.ShapeDtypeStruct((N_NODES, 128), jnp.float32),
    )(acc, den, hr)


def kernel(input_features, edges, W1, a1, R1, W2, a2, W3, a3, R3):
    x = input_features
    tgt = edges[:, 0].astype(jnp.int32)
    src = edges[:, 1].astype(jnp.int32)

    U = UNITS
    w1cat = jnp.concatenate([W1[i] for i in range(H12)], axis=1)
    b1 = jnp.stack([W1[i] @ a1[i, :U] for i in range(H12)]
                   + [W1[i] @ a1[i, U:] for i in range(H12)], axis=1)
    w2cat = jnp.concatenate([W2[i] for i in range(H12)], axis=1)
    b2 = jnp.stack([W2[i] @ a2[i, :U] for i in range(H12)]
                   + [W2[i] @ a2[i, U:] for i in range(H12)], axis=1)
    w3pad = jnp.pad(W3, ((0, 0), (0, 0), (0, 128 - OUT_DIM)))
    w3cat = jnp.concatenate([w3pad[i] for i in range(H3)], axis=1)
    b3 = jnp.stack([W3[i] @ a3[i, :OUT_DIM] for i in range(H3)]
                   + [W3[i] @ a3[i, OUT_DIM:] for i in range(H3)], axis=1)
    r3p = jnp.pad(R3, ((0, 0), (0, 128 - OUT_DIM)))

    hflat1, s1, xr1 = _tck1(x, w1cat, b1, R1)
    acc1, den1 = _sc_edge_4(hflat1.reshape(H12 * N_NODES, 128),
                            s1.T, tgt, src)
    hflat2, s2, h1 = _tck2(acc1, den1.transpose(0, 3, 1, 2), xr1, w2cat, b2)
    acc2, den2 = _sc_edge_4(hflat2.reshape(H12 * N_NODES, 128),
                            s2.T, tgt, src)
    hflat3, s3, hr3 = _tck3(acc2, den2.transpose(0, 3, 1, 2), h1,
                            w3cat, b3, r3p)
    acc3, den3 = _sc_edge_6(hflat3.reshape(H3 * N_NODES, 128),
                            s3.T, tgt, src)
    out = _tck4(acc3, den3.transpose(0, 3, 1, 2), hr3)
    return out[:, :OUT_DIM]


# speedup vs baseline: 7.0102x; 1.4329x over previous
import functools

import jax
import jax.numpy as jnp
from jax import lax
from jax.experimental import pallas as pl
from jax.experimental.pallas import tpu as pltpu
from jax.experimental.pallas import tpu_sc as plsc

N_NODES = 10000
N_EDGES = 320000
D_FEAT = 128
UNITS = 128
H12 = 4
H3 = 6
OUT_DIM = 121

ACC_W = 128
NW = 32
EDGES_PER_W = N_EDGES // NW
CHUNK = 16
EBLK = 2000
NBLK = EDGES_PER_W // EBLK
BLK_CHUNKS = EBLK // CHUNK
ACC_N = 10240
STRIPE = ACC_N // 16
ZROWS = 32

_HIGH = jax.lax.Precision.HIGHEST


def _dot(a, b):
    return jnp.dot(a, b, preferred_element_type=jnp.float32, precision=_HIGH)


def _make_sc_edge(num_heads):
    H = num_heads
    mesh = plsc.VectorSubcoreMesh(core_axis_name="c", subcore_axis_name="s")

    @functools.partial(
        pl.kernel,
        mesh=mesh,
        compiler_params=pltpu.CompilerParams(
            needs_layout_passes=False, use_tc_tiling_on_sc=False),
        out_type=(
            jax.ShapeDtypeStruct((H, 2, N_NODES, ACC_W), jnp.float32),
            jax.ShapeDtypeStruct((H, 2, 16, ACC_N), jnp.float32),
        ),
        scratch_types=[
            pltpu.VMEM((EBLK,), jnp.int32),
            pltpu.VMEM((EBLK,), jnp.int32),
            pltpu.VMEM((N_NODES,), jnp.float32),
            pltpu.VMEM((N_NODES,), jnp.float32),
            pltpu.VMEM((CHUNK, 128), jnp.float32),
            pltpu.VMEM((CHUNK, 128), jnp.float32),
            pltpu.VMEM((CHUNK, 128), jnp.float32),
            pltpu.VMEM((CHUNK, 128), jnp.float32),
            pltpu.VMEM((CHUNK, ACC_W), jnp.float32),
            pltpu.VMEM((CHUNK,), jnp.float32),
            pltpu.VMEM((ZROWS, ACC_W), jnp.float32),
            pltpu.VMEM((ACC_N,), jnp.float32),
            pltpu.VMEM_SHARED((ACC_N, ACC_W), jnp.float32),
            pltpu.SemaphoreType.DMA,
            pltpu.SemaphoreType.DMA,
            pltpu.SemaphoreType.DMA,
            pltpu.SemaphoreType.DMA,
        ],
    )
    def sc_edge(h_hbm, st_hbm, tgt_hbm, src_hbm, out_hbm, den_hbm,
                tgt_v, src_v, st_v, ss_v, grows0, grows1, grows2, grows3,
                rowsbuf, wbuf, zbuf, denom_v, acc, sem0, sem1, sem2, sem3):
        gbufs = (grows0, grows1, grows2, grows3)
        sems = (sem0, sem1, sem2, sem3)
        cid = lax.axis_index("c")
        sid = lax.axis_index("s")
        wid = sid * 2 + cid
        row0 = sid * STRIPE
        zero16 = jnp.zeros((16,), jnp.float32)

        for r in range(ZROWS):
            for j in range(ACC_W // 16):
                zbuf[r, pl.ds(j * 16, 16)] = zero16

        ebase = wid * EDGES_PER_W

        def head_body(h, hcarry):
            pltpu.sync_copy(st_hbm.at[h], st_v)
            pltpu.sync_copy(st_hbm.at[H + h], ss_v)
            for kk in range(STRIPE // ZROWS):
                pltpu.sync_copy(zbuf, acc.at[pl.ds(row0 + kk * ZROWS, ZROWS)])

            def zbody(c, carry):
                denom_v[pl.ds(c * 16, 16)] = zero16
                return carry

            lax.fori_loop(0, ACC_N // 16, zbody, 0)
            plsc.subcore_barrier()

            hoff = h * N_NODES

            def fire(c, gbuf, sem):
                isrc = src_v[pl.ds(c * CHUNK, CHUNK)]
                return pltpu.async_copy(h_hbm.at[isrc + hoff], gbuf, sem)

            def drain(c, gbuf, handle):
                base = c * CHUNK
                it = tgt_v[pl.ds(base, CHUNK)]
                isrc = src_v[pl.ds(base, CHUNK)]
                st = plsc.load_gather(st_v, [it])
                ss = plsc.load_gather(ss_v, [isrc])
                e = st + ss
                e = jnp.where(e >= 0.0, e, 0.2 * e)
                e = jnp.minimum(jnp.maximum(e, -2.0), 2.0)
                w = jnp.exp(e)
                wbuf[...] = w
                plsc.addupdate_scatter(denom_v, [it], w)
                handle.wait()

                def scale_body(i, icarry):
                    wi = plsc.load_gather(
                        wbuf, [jnp.full((16,), 0, jnp.int32) + i])
                    for j in range(8):
                        rowsbuf[i, pl.ds(j * 16, 16)] = (
                            gbuf[i, pl.ds(j * 16, 16)] * wi)
                    return icarry

                lax.fori_loop(0, CHUNK, scale_body, 0)
                pltpu.sync_copy(rowsbuf, acc.at[it], add=True)

            def body(q, carry):
                c0 = q * 4
                handles = [fire(c0 + k, gbufs[k], sems[k]) for k in range(4)]
                for k in range(4):
                    drain(c0 + k, gbufs[k], handles[k])
                return carry

            def blk_body(blk, carry):
                pltpu.sync_copy(
                    tgt_hbm.at[pl.ds(ebase + blk * EBLK, EBLK)], tgt_v)
                pltpu.sync_copy(
                    src_hbm.at[pl.ds(ebase + blk * EBLK, EBLK)], src_v)
                lax.fori_loop(0, BLK_CHUNKS // 4, body, 0)
                last = BLK_CHUNKS - 1
                drain(last, gbufs[0], fire(last, gbufs[0], sems[0]))
                return carry

            lax.fori_loop(0, NBLK, blk_body, 0)
            plsc.subcore_barrier()
            pltpu.sync_copy(denom_v, den_hbm.at[h, cid, sid])

            @pl.when(sid < 15)
            def _():
                pltpu.sync_copy(
                    acc.at[pl.ds(row0, STRIPE)],
                    out_hbm.at[h, cid, pl.ds(row0, STRIPE)])

            @pl.when(sid == 15)
            def _():
                pltpu.sync_copy(
                    acc.at[pl.ds(row0, N_NODES - 15 * STRIPE)],
                    out_hbm.at[h, cid, pl.ds(row0, N_NODES - 15 * STRIPE)])

            plsc.subcore_barrier()
            return hcarry

        lax.fori_loop(0, H, head_body, 0)

    return sc_edge


_sc_edge_4 = _make_sc_edge(H12)
_sc_edge_6 = _make_sc_edge(H3)


NB = 10
BR = N_NODES // NB


def _norm_heads(acc, den):
    num = acc[:, 0] + acc[:, 1]
    d = jnp.sum(den, axis=(2, 3))
    return num / jnp.maximum(d, 1e-20)[..., None]


def _norm_concat(acc, den, res):
    hh = _norm_heads(acc, den)
    cat = jnp.concatenate([hh[i] for i in range(hh.shape[0])], axis=-1)
    return jnp.maximum(cat + res, 0.0)


def _tck1_body(x_ref, wcat_ref, b_ref, r_ref, hflat_ref, s_ref, xr_ref):
    x = x_ref[...]
    y = _dot(x, wcat_ref[...])
    for h in range(H12):
        hflat_ref[h] = y[:, h * 128:(h + 1) * 128]
    s_ref[...] = _dot(x, b_ref[...])
    xr_ref[...] = _dot(x, r_ref[...])


def _tck2_body(acc_ref, den_ref, xr_ref, wcat_ref, b_ref,
               hflat_ref, s_ref, h1_ref):
    h1 = _norm_concat(acc_ref[...], den_ref[...], xr_ref[...])
    y = _dot(h1, wcat_ref[...])
    for h in range(H12):
        hflat_ref[h] = y[:, h * 128:(h + 1) * 128]
    s_ref[...] = _dot(h1, b_ref[...])
    h1_ref[...] = h1


def _tck3_body(acc_ref, den_ref, h1_ref, wcat_ref, b_ref, r3_ref,
               hflat_ref, s_ref, hr_ref):
    h2 = _norm_concat(acc_ref[...], den_ref[...], h1_ref[...])
    y = _dot(h2, wcat_ref[...])
    for h in range(H3):
        hflat_ref[h] = y[:, h * 128:(h + 1) * 128]
    s_ref[...] = _dot(h2, b_ref[...])
    hr_ref[...] = _dot(h2, r3_ref[...])


def _tck4_body(acc_ref, den_ref, hr_ref, out_ref):
    hh = _norm_heads(acc_ref[...], den_ref[...])
    out_ref[...] = jnp.mean(hh, axis=0) + hr_ref[...]


def _row_spec(shape_tail):
    return pl.BlockSpec((BR,) + shape_tail, lambda i: (i,) + (0,) * len(shape_tail))


def _full_spec(shape):
    return pl.BlockSpec(shape, lambda i: (0,) * len(shape))


def _hflat_spec(H):
    return pl.BlockSpec((H, BR, 128), lambda i: (0, i, 0))


def _acc_spec(H):
    return pl.BlockSpec((H, 2, BR, ACC_W), lambda i: (0, 0, i, 0))


def _den_spec(H):
    return pl.BlockSpec((H, BR, 2, 16), lambda i: (0, i, 0, 0))


def _tck1(x, wcat, b, r):
    return pl.pallas_call(
        _tck1_body,
        grid=(NB,),
        in_specs=[_row_spec((D_FEAT,)), _full_spec(wcat.shape),
                  _full_spec(b.shape), _full_spec(r.shape)],
        out_specs=[_hflat_spec(H12), _row_spec((2 * H12,)),
                   _row_spec((H12 * UNITS,))],
        out_shape=[
            jax.ShapeDtypeStruct((H12, N_NODES, 128), jnp.float32),
            jax.ShapeDtypeStruct((N_NODES, 2 * H12), jnp.float32),
            jax.ShapeDtypeStruct((N_NODES, H12 * UNITS), jnp.float32),
        ],
    )(x, wcat, b, r)


def _tck2(acc, den, xr, wcat, b):
    return pl.pallas_call(
        _tck2_body,
        grid=(NB,),
        in_specs=[_acc_spec(H12), _den_spec(H12), _row_spec((H12 * UNITS,)),
                  _full_spec(wcat.shape), _full_spec(b.shape)],
        out_specs=[_hflat_spec(H12), _row_spec((2 * H12,)),
                   _row_spec((H12 * UNITS,))],
        out_shape=[
            jax.ShapeDtypeStruct((H12, N_NODES, 128), jnp.float32),
            jax.ShapeDtypeStruct((N_NODES, 2 * H12), jnp.float32),
            jax.ShapeDtypeStruct((N_NODES, H12 * UNITS), jnp.float32),
        ],
    )(acc, den, xr, wcat, b)


def _tck3(acc, den, h1, wcat, b, r3p):
    return pl.pallas_call(
        _tck3_body,
        grid=(NB,),
        in_specs=[_acc_spec(H12), _den_spec(H12), _row_spec((H12 * UNITS,)),
                  _full_spec(wcat.shape), _full_spec(b.shape),
                  _full_spec(r3p.shape)],
        out_specs=[_hflat_spec(H3), _row_spec((2 * H3,)),
                   _row_spec((128,))],
        out_shape=[
            jax.ShapeDtypeStruct((H3, N_NODES, 128), jnp.float32),
            jax.ShapeDtypeStruct((N_NODES, 2 * H3), jnp.float32),
            jax.ShapeDtypeStruct((N_NODES, 128), jnp.float32),
        ],
    )(acc, den, h1, wcat, b, r3p)


def _tck4(acc, den, hr):
    return pl.pallas_call(
        _tck4_body,
        grid=(NB,),
        in_specs=[_acc_spec(H3), _den_spec(H3), _row_spec((128,))],
        out_specs=pl.BlockSpec((BR, 128), lambda i: (i, 0)),
        out_shape=jax.ShapeDtypeStruct((N_NODES, 128), jnp.float32),
    )(acc, den, hr)


def kernel(input_features, edges, W1, a1, R1, W2, a2, W3, a3, R3):
    x = input_features
    tgt = edges[:, 0].astype(jnp.int32)
    src = edges[:, 1].astype(jnp.int32)

    U = UNITS
    w1cat = jnp.concatenate([W1[i] for i in range(H12)], axis=1)
    b1 = jnp.stack([W1[i] @ a1[i, :U] for i in range(H12)]
                   + [W1[i] @ a1[i, U:] for i in range(H12)], axis=1)
    w2cat = jnp.concatenate([W2[i] for i in range(H12)], axis=1)
    b2 = jnp.stack([W2[i] @ a2[i, :U] for i in range(H12)]
                   + [W2[i] @ a2[i, U:] for i in range(H12)], axis=1)
    w3pad = jnp.pad(W3, ((0, 0), (0, 0), (0, 128 - OUT_DIM)))
    w3cat = jnp.concatenate([w3pad[i] for i in range(H3)], axis=1)
    b3 = jnp.stack([W3[i] @ a3[i, :OUT_DIM] for i in range(H3)]
                   + [W3[i] @ a3[i, OUT_DIM:] for i in range(H3)], axis=1)
    r3p = jnp.pad(R3, ((0, 0), (0, 128 - OUT_DIM)))

    hflat1, s1, xr1 = _tck1(x, w1cat, b1, R1)
    acc1, den1 = _sc_edge_4(hflat1.reshape(H12 * N_NODES, 128),
                            s1.T, tgt, src)
    hflat2, s2, h1 = _tck2(acc1, den1.transpose(0, 3, 1, 2), xr1, w2cat, b2)
    acc2, den2 = _sc_edge_4(hflat2.reshape(H12 * N_NODES, 128),
                            s2.T, tgt, src)
    hflat3, s3, hr3 = _tck3(acc2, den2.transpose(0, 3, 1, 2), h1,
                            w3cat, b3, r3p)
    acc3, den3 = _sc_edge_6(hflat3.reshape(H3 * N_NODES, 128),
                            s3.T, tgt, src)
    out = _tck4(acc3, den3.transpose(0, 3, 1, 2), hr3)
    return out[:, :OUT_DIM]
